# Initial kernel scaffold; baseline (speedup 1.0000x reference)
import jax
import jax.numpy as jnp
from jax.experimental import pallas as pl


def kernel(x, edge_index, batch, g_1, g_2, W_in, b_in, Wc1, bc1, Wc2, bc2, Wq1, bq1, Wq2, bq2, Wq3, bq3, Wk1, bk1, Wk2, bk2, Wk3, bk3, Wv1, bv1, Wv2, bv2, Wv3, bv3, Wfc1, bfc1, Wfc2, bfc2, Wfc3, bfc3):
    raise NotImplementedError("write your pallas kernel here")


# speedup vs baseline: 7.2167x; 7.2167x over previous
import functools

import jax
import jax.numpy as jnp
from jax import lax
from jax.experimental import pallas as pl
from jax.experimental.pallas import tpu as pltpu
from jax.experimental.pallas import tpu_sc as plsc

N = 4096
E = 65536
U = 256
UH = 128
U2 = 512
BN = 128
NBLK = N // BN
EB = 128
EROWS = E // EB
NC = 2
NS = 16
ROWS_PER_TILE = EROWS // NS
RPS = N // NS


@functools.lru_cache(maxsize=None)
def _make_sc_degcount():
    mesh = plsc.VectorSubcoreMesh(core_axis_name="c", subcore_axis_name="s")
    return functools.partial(
        pl.kernel,
        mesh=mesh,
        out_type=jax.ShapeDtypeStruct((NC, N, 16), jnp.float32),
        scratch_types=[
            pltpu.VMEM((ROWS_PER_TILE // 2, EB), jnp.int32),
            pltpu.VMEM((EB, 16), jnp.float32),
            pltpu.VMEM_SHARED((N, 16), jnp.float32),
        ],
    )(_sc_degcount_body)


def _sc_degcount_body(dst_hbm, ones_hbm, zeros_hbm, out_hbm, dst_v, ones_v, acc):
    cid = lax.axis_index("c")
    sid = lax.axis_index("s")
    wid = sid * NC + cid
    nrows = ROWS_PER_TILE // 2
    pltpu.sync_copy(zeros_hbm.at[pl.ds(sid * RPS, RPS)],
                    acc.at[pl.ds(sid * RPS, RPS)])
    pltpu.sync_copy(ones_hbm, ones_v)
    pltpu.sync_copy(dst_hbm.at[pl.ds(wid * nrows, nrows)], dst_v)
    plsc.subcore_barrier()
    for j in range(nrows):
        pltpu.sync_copy(ones_v, acc.at[dst_v.at[j]], add=True)
    plsc.subcore_barrier()
    pltpu.sync_copy(acc.at[pl.ds(sid * RPS, RPS)],
                    out_hbm.at[cid].at[pl.ds(sid * RPS, RPS)])


@functools.lru_cache(maxsize=None)
def _make_sc_segsum():
    mesh = plsc.VectorSubcoreMesh(core_axis_name="c", subcore_axis_name="s")
    return functools.partial(
        pl.kernel,
        mesh=mesh,
        out_type=jax.ShapeDtypeStruct((NC, N, UH), jnp.float32),
        scratch_types=[
            pltpu.VMEM((1, EB), jnp.int32),
            pltpu.VMEM((1, EB), jnp.int32),
            pltpu.VMEM((EB, UH), jnp.float32),
            pltpu.VMEM_SHARED((N, UH), jnp.float32),
            pltpu.SemaphoreType.DMA,
        ],
    )(_sc_segsum_body)


def _sc_segsum_body(feat2_hbm, src_hbm, dst_hbm, zeros_hbm, out_hbm,
                    src_v, dst_v, buf, acc, sem):
    cid = lax.axis_index("c")
    sid = lax.axis_index("s")
    pltpu.sync_copy(zeros_hbm.at[pl.ds(sid * RPS, RPS)],
                    acc.at[pl.ds(sid * RPS, RPS)])
    plsc.subcore_barrier()
    base = sid * ROWS_PER_TILE

    def step(j, carry):
        pltpu.sync_copy(src_hbm.at[pl.ds(base + j, 1)], src_v)
        pltpu.sync_copy(dst_hbm.at[pl.ds(base + j, 1)], dst_v)
        pltpu.async_copy(feat2_hbm.at[cid].at[src_v.at[0]], buf, sem).wait()
        pltpu.sync_copy(buf, acc.at[dst_v.at[0]], add=True)
        return carry

    lax.fori_loop(0, ROWS_PER_TILE, step, 0)
    plsc.subcore_barrier()
    pltpu.sync_copy(acc.at[pl.ds(sid * RPS, RPS)],
                    out_hbm.at[cid].at[pl.ds(sid * RPS, RPS)])


def _split_store(out_ref, t):
    out_ref[0] = t[:, :UH]
    out_ref[1] = t[:, UH:]


def _prep_body(cnt_ref, xp_ref, win_ref, dinv_ref, xw0s_ref):
    cnt = cnt_ref[...]
    deg = cnt[0, :, 0:1] + cnt[1, :, 0:1] + 1.0
    dinv = lax.rsqrt(deg)
    dinv_ref[...] = jnp.broadcast_to(dinv, (N, 128))
    xw = jnp.dot(xp_ref[...], win_ref[...], preferred_element_type=jnp.float32)
    _split_store(xw0s_ref, dinv * xw)


def _attn_body(p_ref, xs_ref, dinv_ref, bin_ref,
               wq1, bq1, wq2, bq2, wq3, bq3,
               wk1, bk1, wk2, bk2, wk3, bk3,
               wv1, bv1, wv2, bv2, wv3, bv3,
               wc1, out_ref):
    dinv = dinv_ref[:, 0:1]
    p = jnp.concatenate([p_ref[0], p_ref[1]], axis=1)
    xs = jnp.concatenate([xs_ref[0], xs_ref[1]], axis=1)
    h0 = jnp.tanh(dinv * (p + xs) + bin_ref[...])

    def mlp(h, w1, b1, w2, b2, w3, b3):
        h = jnp.maximum(jnp.dot(h, w1[...], preferred_element_type=jnp.float32)
                        + b1[...], 0.0)
        h = jnp.maximum(jnp.dot(h, w2[...], preferred_element_type=jnp.float32)
                        + b2[...], 0.0)
        return jnp.maximum(jnp.dot(h, w3[...], preferred_element_type=jnp.float32)
                           + b3[...], 0.0)

    q = mlp(h0, wq1, bq1, wq2, bq2, wq3, bq3)
    k = mlp(h0, wk1, bk1, wk2, bk2, wk3, bk3)
    v = mlp(h0, wv1, bv1, wv2, bv2, wv3, bv3)
    s = jnp.dot(q, k.T, preferred_element_type=jnp.float32) * (1.0 / 16.0)
    m = jnp.max(s, axis=-1, keepdims=True)
    e = jnp.exp(s - m)
    pr = e / jnp.sum(e, axis=-1, keepdims=True)
    att = jnp.dot(pr, v, preferred_element_type=jnp.float32)
    _split_store(out_ref,
                 dinv * jnp.dot(att, wc1[...],
                                preferred_element_type=jnp.float32))


def _mid_body(p_ref, hw_ref, dinv_ref, bc1_ref, wc2_ref, out_ref):
    dinv = dinv_ref[:, 0:1]
    p = jnp.concatenate([p_ref[0], p_ref[1]], axis=1)
    hw = jnp.concatenate([hw_ref[0], hw_ref[1]], axis=1)
    h1 = jnp.tanh(dinv * (p + hw) + bc1_ref[...])
    _split_store(out_ref,
                 dinv * jnp.dot(h1, wc2_ref[...],
                                preferred_element_type=jnp.float32))


def _final_body(p_ref, hw_ref, dinv_ref, bc2_ref,
                wfc1, bfc1, wfc2, bfc2, w3row_ref, b3_ref, out_ref):
    dinv = dinv_ref[:, 0:1]
    p = jnp.concatenate([p_ref[0], p_ref[1]], axis=1)
    hw = jnp.concatenate([hw_ref[0], hw_ref[1]], axis=1)
    h2 = jnp.tanh(dinv * (p + hw) + bc2_ref[...])
    f = jnp.tanh(jnp.dot(h2, wfc1[...], preferred_element_type=jnp.float32)
                 + bfc1[...])
    f = jnp.tanh(jnp.dot(f, wfc2[...], preferred_element_type=jnp.float32)
                 + bfc2[...])
    g = jnp.sum(f * w3row_ref[...], axis=1)
    pooled = jnp.mean(g) + b3_ref[0, 0]
    out_ref[...] = jnp.broadcast_to(jax.nn.sigmoid(pooled), (1, 1, 128))


def _tc_prep(cnt, xp, winp):
    return pl.pallas_call(
        _prep_body,
        out_shape=(jax.ShapeDtypeStruct((N, 128), jnp.float32),
                   jax.ShapeDtypeStruct((NC, N, UH), jnp.float32)),
    )(cnt, xp, winp)


def _full(a):
    return pl.BlockSpec(a.shape, lambda b: (0,) * a.ndim)


def _tc_attn(p0, xw0s, dinv128, bin2, ws):
    wspecs = [_full(w) for w in ws]
    return pl.pallas_call(
        _attn_body,
        grid=(NBLK,),
        in_specs=[
            pl.BlockSpec((NC, BN, UH), lambda b: (0, b, 0)),
            pl.BlockSpec((NC, BN, UH), lambda b: (0, b, 0)),
            pl.BlockSpec((BN, 128), lambda b: (b, 0)),
            pl.BlockSpec((1, U), lambda b: (0, 0)),
        ] + wspecs,
        out_specs=pl.BlockSpec((NC, BN, UH), lambda b: (0, b, 0)),
        out_shape=jax.ShapeDtypeStruct((NC, N, UH), jnp.float32),
    )(p0, xw0s, dinv128, bin2, *ws)


def _tc_mid(p1, hw1s, dinv128, bc12, wc2):
    R = 256
    return pl.pallas_call(
        _mid_body,
        grid=(N // R,),
        in_specs=[
            pl.BlockSpec((NC, R, UH), lambda b: (0, b, 0)),
            pl.BlockSpec((NC, R, UH), lambda b: (0, b, 0)),
            pl.BlockSpec((R, 128), lambda b: (b, 0)),
            pl.BlockSpec((1, U), lambda b: (0, 0)),
            pl.BlockSpec((U, U), lambda b: (0, 0)),
        ],
        out_specs=pl.BlockSpec((NC, R, UH), lambda b: (0, b, 0)),
        out_shape=jax.ShapeDtypeStruct((NC, N, UH), jnp.float32),
    )(p1, hw1s, dinv128, bc12, wc2)


def _tc_final(p2, hw2s, dinv128, bc22, wfc1, bfc12, wfc2, bfc22, w3row, b3):
    return pl.pallas_call(
        _final_body,
        grid=(NBLK,),
        in_specs=[
            pl.BlockSpec((NC, BN, UH), lambda b: (0, b, 0)),
            pl.BlockSpec((NC, BN, UH), lambda b: (0, b, 0)),
            pl.BlockSpec((BN, 128), lambda b: (b, 0)),
            pl.BlockSpec((1, U), lambda b: (0, 0)),
            pl.BlockSpec((U, U), lambda b: (0, 0)),
            pl.BlockSpec((1, U), lambda b: (0, 0)),
            pl.BlockSpec((U, 32), lambda b: (0, 0)),
            pl.BlockSpec((1, 32), lambda b: (0, 0)),
            pl.BlockSpec((1, 32), lambda b: (0, 0)),
            pl.BlockSpec((1, 128), lambda b: (0, 0)),
        ],
        out_specs=pl.BlockSpec((1, 1, 128), lambda b: (b, 0, 0)),
        out_shape=jax.ShapeDtypeStruct((NBLK, 1, 128), jnp.float32),
    )(p2, hw2s, dinv128, bc22, wfc1, bfc12, wfc2, bfc22, w3row, b3)


def kernel(x, edge_index, batch, g_1, g_2, W_in, b_in, Wc1, bc1, Wc2, bc2,
           Wq1, bq1, Wq2, bq2, Wq3, bq3, Wk1, bk1, Wk2, bk2, Wk3, bk3,
           Wv1, bv1, Wv2, bv2, Wv3, bv3, Wfc1, bfc1, Wfc2, bfc2, Wfc3, bfc3):
    src = edge_index[0].reshape(EROWS, EB)
    dst = edge_index[1].reshape(EROWS, EB)
    xp = jnp.pad(x, ((0, 0), (0, 7)))
    winp = jnp.pad(W_in, ((0, 7), (0, 0)))
    zeros_nh = jnp.zeros((N, UH), jnp.float32)
    zeros_n16 = jnp.zeros((N, 16), jnp.float32)
    ones_e16 = jnp.ones((EB, 16), jnp.float32)

    cnt = _make_sc_degcount()(dst, ones_e16, zeros_n16)
    dinv128, xw0s = _tc_prep(cnt, xp, winp)

    segsum = _make_sc_segsum()
    p0 = segsum(xw0s, src, dst, zeros_nh)
    ws = (Wq1, bq1.reshape(1, -1), Wq2, bq2.reshape(1, -1),
          Wq3, bq3.reshape(1, -1),
          Wk1, bk1.reshape(1, -1), Wk2, bk2.reshape(1, -1),
          Wk3, bk3.reshape(1, -1),
          Wv1, bv1.reshape(1, -1), Wv2, bv2.reshape(1, -1),
          Wv3, bv3.reshape(1, -1), Wc1)
    hw1s = _tc_attn(p0, xw0s, dinv128, b_in.reshape(1, -1), ws)

    p1 = segsum(hw1s, src, dst, zeros_nh)
    hw2s = _tc_mid(p1, hw1s, dinv128, bc1.reshape(1, -1), Wc2)

    p2 = segsum(hw2s, src, dst, zeros_nh)
    outp = _tc_final(p2, hw2s, dinv128, bc2.reshape(1, -1),
                     Wfc1, bfc1.reshape(1, -1), Wfc2, bfc2.reshape(1, -1),
                     Wfc3.reshape(1, -1),
                     jnp.broadcast_to(bfc3.reshape(1, 1), (1, 128)))
    return outp[:, 0, :1]


# speedup vs baseline: 10.4060x; 1.4419x over previous
import functools

import jax
import jax.numpy as jnp
from jax import lax
from jax.experimental import pallas as pl
from jax.experimental.pallas import tpu as pltpu
from jax.experimental.pallas import tpu_sc as plsc

N = 4096
E = 65536
U = 256
UH = 128
U2 = 512
BN = 128
NBLK = N // BN
EB = 128
EROWS = E // EB
NC = 2
NS = 16
ROWS_PER_TILE = EROWS // NS
RPS = N // NS


@functools.lru_cache(maxsize=None)
def _make_sc_degcount():
    mesh = plsc.VectorSubcoreMesh(core_axis_name="c", subcore_axis_name="s")
    return functools.partial(
        pl.kernel,
        mesh=mesh,
        out_type=jax.ShapeDtypeStruct((NC, N, 16), jnp.float32),
        scratch_types=[
            pltpu.VMEM((ROWS_PER_TILE // 2, EB), jnp.int32),
            pltpu.VMEM((EB, 16), jnp.float32),
            pltpu.VMEM_SHARED((N, 16), jnp.float32),
        ],
    )(_sc_degcount_body)


def _sc_degcount_body(dst_hbm, ones_hbm, zeros_hbm, out_hbm, dst_v, ones_v, acc):
    cid = lax.axis_index("c")
    sid = lax.axis_index("s")
    wid = sid * NC + cid
    nrows = ROWS_PER_TILE // 2
    pltpu.sync_copy(zeros_hbm.at[pl.ds(sid * RPS, RPS)],
                    acc.at[pl.ds(sid * RPS, RPS)])
    pltpu.sync_copy(ones_hbm, ones_v)
    pltpu.sync_copy(dst_hbm.at[pl.ds(wid * nrows, nrows)], dst_v)
    plsc.subcore_barrier()
    for j in range(nrows):
        pltpu.sync_copy(ones_v, acc.at[dst_v.at[j]], add=True)
    plsc.subcore_barrier()
    pltpu.sync_copy(acc.at[pl.ds(sid * RPS, RPS)],
                    out_hbm.at[cid].at[pl.ds(sid * RPS, RPS)])


@functools.lru_cache(maxsize=None)
def _make_sc_segsum():
    mesh = plsc.VectorSubcoreMesh(core_axis_name="c", subcore_axis_name="s")
    return functools.partial(
        pl.kernel,
        mesh=mesh,
        out_type=jax.ShapeDtypeStruct((NC, N, UH), jnp.float32),
        scratch_types=[
            pltpu.VMEM((ROWS_PER_TILE, EB), jnp.int32),
            pltpu.VMEM((ROWS_PER_TILE, EB), jnp.int32),
            pltpu.VMEM((EB, UH), jnp.float32),
            pltpu.VMEM((EB, UH), jnp.float32),
            pltpu.VMEM_SHARED((N, UH), jnp.float32),
            pltpu.SemaphoreType.DMA,
            pltpu.SemaphoreType.DMA,
        ],
    )(_sc_segsum_body)


def _sc_segsum_body(feat2_hbm, src_hbm, dst_hbm, zeros_hbm, out_hbm,
                    src_v, dst_v, buf0, buf1, acc, sem0, sem1):
    cid = lax.axis_index("c")
    sid = lax.axis_index("s")
    pltpu.sync_copy(zeros_hbm.at[pl.ds(sid * RPS, RPS)],
                    acc.at[pl.ds(sid * RPS, RPS)])
    base = sid * ROWS_PER_TILE
    pltpu.sync_copy(src_hbm.at[pl.ds(base, ROWS_PER_TILE)], src_v)
    pltpu.sync_copy(dst_hbm.at[pl.ds(base, ROWS_PER_TILE)], dst_v)
    plsc.subcore_barrier()

    feat = feat2_hbm.at[cid]
    pltpu.async_copy(feat.at[src_v.at[0]], buf0, sem0)

    def step(i, carry):
        j0, j1 = 2 * i, 2 * i + 1
        pltpu.make_async_copy(feat.at[src_v.at[0]], buf0, sem0).wait()
        pltpu.async_copy(feat.at[src_v.at[j1]], buf1, sem1)
        pltpu.sync_copy(buf0, acc.at[dst_v.at[j0]], add=True)
        pltpu.make_async_copy(feat.at[src_v.at[0]], buf1, sem1).wait()
        jn = jnp.minimum(j0 + 2, ROWS_PER_TILE - 1)
        pltpu.async_copy(feat.at[src_v.at[jn]], buf0, sem0)
        pltpu.sync_copy(buf1, acc.at[dst_v.at[j1]], add=True)
        return carry

    lax.fori_loop(0, ROWS_PER_TILE // 2, step, 0)
    pltpu.make_async_copy(feat.at[src_v.at[0]], buf0, sem0).wait()
    plsc.subcore_barrier()
    pltpu.sync_copy(acc.at[pl.ds(sid * RPS, RPS)],
                    out_hbm.at[cid].at[pl.ds(sid * RPS, RPS)])


def _split_store(out_ref, t):
    out_ref[0] = t[:, :UH]
    out_ref[1] = t[:, UH:]


def _prep_body(cnt_ref, xp_ref, win_ref, dinv_ref, xw0s_ref):
    cnt = cnt_ref[...]
    deg = cnt[0, :, 0:1] + cnt[1, :, 0:1] + 1.0
    dinv = lax.rsqrt(deg)
    dinv_ref[...] = jnp.broadcast_to(dinv, (N, 128))
    xw = jnp.dot(xp_ref[...], win_ref[...], preferred_element_type=jnp.float32)
    _split_store(xw0s_ref, dinv * xw)


def _attn_body(p_ref, xs_ref, dinv_ref, bin_ref,
               wq1, bq1, wq2, bq2, wq3, bq3,
               wk1, bk1, wk2, bk2, wk3, bk3,
               wv1, bv1, wv2, bv2, wv3, bv3,
               wc1, out_ref):
    dinv = dinv_ref[:, 0:1]
    p = jnp.concatenate([p_ref[0], p_ref[1]], axis=1)
    xs = jnp.concatenate([xs_ref[0], xs_ref[1]], axis=1)
    h0 = jnp.tanh(dinv * (p + xs) + bin_ref[...])

    def mlp(h, w1, b1, w2, b2, w3, b3):
        h = jnp.maximum(jnp.dot(h, w1[...], preferred_element_type=jnp.float32)
                        + b1[...], 0.0)
        h = jnp.maximum(jnp.dot(h, w2[...], preferred_element_type=jnp.float32)
                        + b2[...], 0.0)
        return jnp.maximum(jnp.dot(h, w3[...], preferred_element_type=jnp.float32)
                           + b3[...], 0.0)

    q = mlp(h0, wq1, bq1, wq2, bq2, wq3, bq3)
    k = mlp(h0, wk1, bk1, wk2, bk2, wk3, bk3)
    v = mlp(h0, wv1, bv1, wv2, bv2, wv3, bv3)
    s = jnp.dot(q, k.T, preferred_element_type=jnp.float32) * (1.0 / 16.0)
    m = jnp.max(s, axis=-1, keepdims=True)
    e = jnp.exp(s - m)
    pr = e / jnp.sum(e, axis=-1, keepdims=True)
    att = jnp.dot(pr, v, preferred_element_type=jnp.float32)
    _split_store(out_ref,
                 dinv * jnp.dot(att, wc1[...],
                                preferred_element_type=jnp.float32))


def _mid_body(p_ref, hw_ref, dinv_ref, bc1_ref, wc2_ref, out_ref):
    dinv = dinv_ref[:, 0:1]
    p = jnp.concatenate([p_ref[0], p_ref[1]], axis=1)
    hw = jnp.concatenate([hw_ref[0], hw_ref[1]], axis=1)
    h1 = jnp.tanh(dinv * (p + hw) + bc1_ref[...])
    _split_store(out_ref,
                 dinv * jnp.dot(h1, wc2_ref[...],
                                preferred_element_type=jnp.float32))


def _final_body(p_ref, hw_ref, dinv_ref, bc2_ref,
                wfc1, bfc1, wfc2, bfc2, w3row_ref, b3_ref, out_ref):
    dinv = dinv_ref[:, 0:1]
    p = jnp.concatenate([p_ref[0], p_ref[1]], axis=1)
    hw = jnp.concatenate([hw_ref[0], hw_ref[1]], axis=1)
    h2 = jnp.tanh(dinv * (p + hw) + bc2_ref[...])
    f = jnp.tanh(jnp.dot(h2, wfc1[...], preferred_element_type=jnp.float32)
                 + bfc1[...])
    f = jnp.tanh(jnp.dot(f, wfc2[...], preferred_element_type=jnp.float32)
                 + bfc2[...])
    g = jnp.sum(f * w3row_ref[...], axis=1)
    pooled = jnp.mean(g) + b3_ref[0, 0]
    out_ref[...] = jnp.broadcast_to(jax.nn.sigmoid(pooled), (1, 1, 128))


def _tc_prep(cnt, xp, winp):
    return pl.pallas_call(
        _prep_body,
        out_shape=(jax.ShapeDtypeStruct((N, 128), jnp.float32),
                   jax.ShapeDtypeStruct((NC, N, UH), jnp.float32)),
    )(cnt, xp, winp)


def _full(a):
    return pl.BlockSpec(a.shape, lambda b: (0,) * a.ndim)


def _tc_attn(p0, xw0s, dinv128, bin2, ws):
    wspecs = [_full(w) for w in ws]
    return pl.pallas_call(
        _attn_body,
        grid=(NBLK,),
        in_specs=[
            pl.BlockSpec((NC, BN, UH), lambda b: (0, b, 0)),
            pl.BlockSpec((NC, BN, UH), lambda b: (0, b, 0)),
            pl.BlockSpec((BN, 128), lambda b: (b, 0)),
            pl.BlockSpec((1, U), lambda b: (0, 0)),
        ] + wspecs,
        out_specs=pl.BlockSpec((NC, BN, UH), lambda b: (0, b, 0)),
        out_shape=jax.ShapeDtypeStruct((NC, N, UH), jnp.float32),
    )(p0, xw0s, dinv128, bin2, *ws)


def _tc_mid(p1, hw1s, dinv128, bc12, wc2):
    R = 256
    return pl.pallas_call(
        _mid_body,
        grid=(N // R,),
        in_specs=[
            pl.BlockSpec((NC, R, UH), lambda b: (0, b, 0)),
            pl.BlockSpec((NC, R, UH), lambda b: (0, b, 0)),
            pl.BlockSpec((R, 128), lambda b: (b, 0)),
            pl.BlockSpec((1, U), lambda b: (0, 0)),
            pl.BlockSpec((U, U), lambda b: (0, 0)),
        ],
        out_specs=pl.BlockSpec((NC, R, UH), lambda b: (0, b, 0)),
        out_shape=jax.ShapeDtypeStruct((NC, N, UH), jnp.float32),
    )(p1, hw1s, dinv128, bc12, wc2)


def _tc_final(p2, hw2s, dinv128, bc22, wfc1, bfc12, wfc2, bfc22, w3row, b3):
    return pl.pallas_call(
        _final_body,
        grid=(NBLK,),
        in_specs=[
            pl.BlockSpec((NC, BN, UH), lambda b: (0, b, 0)),
            pl.BlockSpec((NC, BN, UH), lambda b: (0, b, 0)),
            pl.BlockSpec((BN, 128), lambda b: (b, 0)),
            pl.BlockSpec((1, U), lambda b: (0, 0)),
            pl.BlockSpec((U, U), lambda b: (0, 0)),
            pl.BlockSpec((1, U), lambda b: (0, 0)),
            pl.BlockSpec((U, 32), lambda b: (0, 0)),
            pl.BlockSpec((1, 32), lambda b: (0, 0)),
            pl.BlockSpec((1, 32), lambda b: (0, 0)),
            pl.BlockSpec((1, 128), lambda b: (0, 0)),
        ],
        out_specs=pl.BlockSpec((1, 1, 128), lambda b: (b, 0, 0)),
        out_shape=jax.ShapeDtypeStruct((NBLK, 1, 128), jnp.float32),
    )(p2, hw2s, dinv128, bc22, wfc1, bfc12, wfc2, bfc22, w3row, b3)


def kernel(x, edge_index, batch, g_1, g_2, W_in, b_in, Wc1, bc1, Wc2, bc2,
           Wq1, bq1, Wq2, bq2, Wq3, bq3, Wk1, bk1, Wk2, bk2, Wk3, bk3,
           Wv1, bv1, Wv2, bv2, Wv3, bv3, Wfc1, bfc1, Wfc2, bfc2, Wfc3, bfc3):
    src = edge_index[0].reshape(EROWS, EB)
    dst = edge_index[1].reshape(EROWS, EB)
    xp = jnp.pad(x, ((0, 0), (0, 7)))
    winp = jnp.pad(W_in, ((0, 7), (0, 0)))
    zeros_nh = jnp.zeros((N, UH), jnp.float32)
    zeros_n16 = jnp.zeros((N, 16), jnp.float32)
    ones_e16 = jnp.ones((EB, 16), jnp.float32)

    cnt = _make_sc_degcount()(dst, ones_e16, zeros_n16)
    dinv128, xw0s = _tc_prep(cnt, xp, winp)

    segsum = _make_sc_segsum()
    p0 = segsum(xw0s, src, dst, zeros_nh)
    ws = (Wq1, bq1.reshape(1, -1), Wq2, bq2.reshape(1, -1),
          Wq3, bq3.reshape(1, -1),
          Wk1, bk1.reshape(1, -1), Wk2, bk2.reshape(1, -1),
          Wk3, bk3.reshape(1, -1),
          Wv1, bv1.reshape(1, -1), Wv2, bv2.reshape(1, -1),
          Wv3, bv3.reshape(1, -1), Wc1)
    hw1s = _tc_attn(p0, xw0s, dinv128, b_in.reshape(1, -1), ws)

    p1 = segsum(hw1s, src, dst, zeros_nh)
    hw2s = _tc_mid(p1, hw1s, dinv128, bc1.reshape(1, -1), Wc2)

    p2 = segsum(hw2s, src, dst, zeros_nh)
    outp = _tc_final(p2, hw2s, dinv128, bc2.reshape(1, -1),
                     Wfc1, bfc1.reshape(1, -1), Wfc2, bfc2.reshape(1, -1),
                     Wfc3.reshape(1, -1),
                     jnp.broadcast_to(bfc3.reshape(1, 1), (1, 128)))
    return outp[:, 0, :1]
